# baseline (speedup 1.0000x reference)
import functools

import jax
import jax.numpy as jnp
from jax import lax
from jax.experimental import pallas as pl
from jax.experimental.pallas import tpu as pltpu
from jax.experimental.pallas import tpu_sc as plsc

HID = 768
L = 16
NJ = HID // L
C = 32
NG = C // L
CSPAN = 64 * HID - (NJ - 1) * L
EPS = 1e-6

_GATHER_DNUMS = lax.GatherDimensionNumbers(
    offset_dims=(), collapsed_slice_dims=(0,), start_index_map=(0,))


def _lane_splat(vec16, lane):
    idx = jnp.full((L,), lane, jnp.int32)
    return lax.gather(vec16, idx[:, None], _GATHER_DNUMS, (1,),
                      mode=lax.GatherScatterMode.PROMISE_IN_BOUNDS)


def _rsqrt16(v):
    i = plsc.bitcast(v, jnp.int32)
    i = jnp.int32(0x5F3759DF) - (i >> 1)
    y = plsc.bitcast(i, jnp.float32)
    for _ in range(3):
        y = y * (1.5 - 0.5 * v * y * y)
    return y


def _make_sc_kernel(n_tokens):
    info = plsc.get_sparse_core_info()
    nw = info.num_cores * info.num_subcores
    tpw = n_tokens // nw
    nchunk = tpw // C
    npair = nchunk // 2

    mesh = plsc.VectorSubcoreMesh(core_axis_name="c", subcore_axis_name="s")

    @functools.partial(
        pl.kernel,
        out_type=jax.ShapeDtypeStruct((n_tokens, HID), jnp.float32),
        mesh=mesh,
        compiler_params=pltpu.CompilerParams(needs_layout_passes=False),
        scratch_types=[
            pltpu.VMEM((64 * HID,), jnp.float32),
            pltpu.VMEM((HID,), jnp.float32),
            pltpu.VMEM((C,), jnp.int32),
            pltpu.VMEM((C,), jnp.int32),
            pltpu.VMEM((C,), jnp.int32),
            pltpu.VMEM((C,), jnp.int32),
            pltpu.VMEM((C,), jnp.int32),
            pltpu.VMEM((C,), jnp.int32),
            pltpu.VMEM((C,), jnp.int32),
            pltpu.VMEM((C,), jnp.int32),
            pltpu.VMEM((C,), jnp.int32),
            pltpu.VMEM((C, HID), jnp.float32),
            pltpu.VMEM((C, HID), jnp.float32),
            pltpu.VMEM((C, HID), jnp.float32),
            pltpu.SemaphoreType.DMA,
            pltpu.SemaphoreType.DMA,
            pltpu.SemaphoreType.DMA,
            pltpu.SemaphoreType.DMA,
            pltpu.SemaphoreType.DMA,
            pltpu.SemaphoreType.DMA,
            pltpu.SemaphoreType.DMA,
            pltpu.SemaphoreType.DMA,
            pltpu.SemaphoreType.DMA,
            pltpu.SemaphoreType.DMA,
            pltpu.SemaphoreType.DMA,
            pltpu.SemaphoreType.DMA,
        ],
    )
    def sc_kernel(ids_h, tid_h, kid_h, word_h, tok_h, task_h, w_h, out_h,
                  combo_v, w_v,
                  idx0, idx1, idx2, tid0, tid1, tid2, kid0, kid1, kid2,
                  rows0, rows1, rows2,
                  i0, i1, i2, tk0, tk1, tk2, g0, g1, g2, o0, o1, o2):
        cid = lax.axis_index("c")
        sid = lax.axis_index("s")
        wid = sid * info.num_cores + cid
        base = wid * tpw

        iota = lax.iota(jnp.int32, L)

        idx_b = (idx0, idx1, idx2)
        tid_b = (tid0, tid1, tid2)
        kid_b = (kid0, kid1, kid2)
        rows_b = (rows0, rows1, rows2)
        i_sem = (i0, i1, i2)
        tk_sem = (tk0, tk1, tk2)
        g_sem = (g0, g1, g2)
        o_sem = (o0, o1, o2)

        def start_in(tb, b):
            pltpu.async_copy(ids_h.at[pl.ds(tb, C)], idx_b[b], i_sem[b])
            pltpu.async_copy(tid_h.at[pl.ds(tb, C)], tid_b[b], tk_sem[b])
            pltpu.async_copy(kid_h.at[pl.ds(tb, C)], kid_b[b], tk_sem[b])

        def wait_in_ids(b):
            pltpu.make_async_copy(ids_h.at[pl.ds(0, C)], idx_b[b],
                                  i_sem[b]).wait()

        def wait_in_tk(b):
            pltpu.make_async_copy(tid_h.at[pl.ds(0, C)], tid_b[b],
                                  tk_sem[b]).wait()
            pltpu.make_async_copy(kid_h.at[pl.ds(0, C)], kid_b[b],
                                  tk_sem[b]).wait()

        def start_gather(b):
            pltpu.async_copy(word_h.at[idx_b[b]], rows_b[b], g_sem[b])

        def wait_gather(b):
            pltpu.make_async_copy(word_h.at[idx_b[b]], rows_b[b],
                                  g_sem[b]).wait()

        def start_out(tb, b):
            pltpu.async_copy(rows_b[b], out_h.at[pl.ds(tb, C)], o_sem[b])

        def wait_out(b):
            pltpu.make_async_copy(rows_b[b], out_h.at[pl.ds(0, C)],
                                  o_sem[b]).wait()

        def compute_group(b, g):
            rows_v = rows_b[b]
            tid16 = tid_b[b][pl.ds(g * L, L)]
            kid16 = kid_b[b][pl.ds(g * L, L)]
            c16 = tid16 * 16 + kid16
            zero = jnp.zeros((L,), jnp.float32)
            PJ = 8
            sums = []
            for half in range(L // PJ):
                t0 = half * PJ
                cbs = [_lane_splat(c16, t0 + t) * HID + iota
                       for t in range(PJ)]

                @plsc.parallel_loop(0, NJ, unroll=2,
                                    carry=tuple([zero] * PJ))
                def _p1(j, accs):
                    off = j * L
                    combo_s = combo_v.at[pl.ds(off, CSPAN)]
                    out = []
                    for t in range(PJ):
                        w = rows_v[g * L + t0 + t, pl.ds(off, L)]
                        cv = plsc.load_gather(combo_s, [cbs[t]])
                        x = w + cv
                        rows_v[g * L + t0 + t, pl.ds(off, L)] = x
                        out.append(accs[t] + x * x)
                    return tuple(out)

                sums.extend(_p1)

            var = zero
            for t in range(L):
                var = jnp.where(iota == t, jnp.sum(sums[t]), var)
            r16 = _rsqrt16(var * (1.0 / HID) + EPS)

            @plsc.parallel_loop(0, NJ, unroll=2)
            def _scale(j):
                wj = w_v[pl.ds(j * L, L)]
                for t16 in range(L):
                    t = g * L + t16
                    r = _lane_splat(r16, t16)
                    rows_v[t, pl.ds(j * L, L)] = (
                        rows_v[t, pl.ds(j * L, L)] * r * wj)

        start_in(base, 0)
        start_in(base + C, 1)
        start_in(base + 2 * C, 2)
        pltpu.sync_copy(w_h, w_v)
        pltpu.sync_copy(tok_h, rows2.at[pl.ds(0, 4)])
        pltpu.sync_copy(task_h, rows2.at[pl.ds(8, 16)])
        wait_in_ids(0)
        start_gather(0)
        wait_in_ids(1)
        start_gather(1)

        @plsc.parallel_loop(0, 4 * 16, unroll=2)
        def _build(cc):
            rt = cc >> 4
            rk = 8 + (cc & 15)
            bc = cc * HID
            for j in range(NJ):
                combo_v[pl.ds(bc + j * L, L)] = (
                    rows2[rt, pl.ds(j * L, L)] + rows2[rk, pl.ds(j * L, L)])

        def chunk_body(n_tb, b, first, gather_ok, in_ok):
            bnn = (b + 2) % 3
            wait_gather(b)
            wait_in_tk(b)
            compute_group(b, 0)

            @pl.when(jnp.logical_not(first))
            def _():
                wait_out(bnn)

            @pl.when(gather_ok)
            def _():
                wait_in_ids(bnn)
                start_gather(bnn)

            compute_group(b, 1)
            start_out(n_tb, b)

            @pl.when(in_ok)
            def _():
                start_in(n_tb + 3 * C, b)

        @pl.loop(0, (nchunk - 1) // 3)
        def _triple(p):
            tb0 = base + (3 * p) * C
            chunk_body(tb0, 0, p == 0, True, True)
            chunk_body(tb0 + C, 1, False, True, p < 4)
            chunk_body(tb0 + 2 * C, 2, False, p < 4, p < 4)

        chunk_body(base + (nchunk - 1) * C, 0, False, False, False)
        wait_out(0)

    return sc_kernel


def kernel(input_ids, token_type_ids, task_type_ids, word_table,
           token_type_table, task_type_table, rms_weight):
    b, s = input_ids.shape
    n = b * s
    out = _make_sc_kernel(n)(
        input_ids.reshape(n),
        token_type_ids.reshape(n),
        task_type_ids.reshape(n),
        word_table,
        token_type_table,
        task_type_table,
        rms_weight,
    )
    return out.reshape(b, s, HID)


# speedup vs baseline: 1.0105x; 1.0105x over previous
import functools

import jax
import jax.numpy as jnp
from jax import lax
from jax.experimental import pallas as pl
from jax.experimental.pallas import tpu as pltpu
from jax.experimental.pallas import tpu_sc as plsc

HID = 768
L = 16
NJ = HID // L
C = 32
NG = C // L
CSPAN = 64 * HID - (NJ - 1) * L
EPS = 1e-6

_GATHER_DNUMS = lax.GatherDimensionNumbers(
    offset_dims=(), collapsed_slice_dims=(0,), start_index_map=(0,))


def _lane_splat(vec16, lane):
    idx = jnp.full((L,), lane, jnp.int32)
    return lax.gather(vec16, idx[:, None], _GATHER_DNUMS, (1,),
                      mode=lax.GatherScatterMode.PROMISE_IN_BOUNDS)


def _rsqrt16(v):
    i = plsc.bitcast(v, jnp.int32)
    i = jnp.int32(0x5F3759DF) - (i >> 1)
    y = plsc.bitcast(i, jnp.float32)
    for _ in range(3):
        y = y * (1.5 - 0.5 * v * y * y)
    return y


def _make_sc_kernel(n_tokens):
    info = plsc.get_sparse_core_info()
    nw = info.num_cores * info.num_subcores
    tpw = n_tokens // nw
    nchunk = tpw // C
    npair = nchunk // 2

    mesh = plsc.VectorSubcoreMesh(core_axis_name="c", subcore_axis_name="s")

    @functools.partial(
        pl.kernel,
        out_type=jax.ShapeDtypeStruct((n_tokens, HID), jnp.float32),
        mesh=mesh,
        compiler_params=pltpu.CompilerParams(needs_layout_passes=False),
        scratch_types=[
            pltpu.VMEM((4 * HID,), jnp.float32),
            pltpu.VMEM((16 * HID,), jnp.float32),
            pltpu.VMEM((64 * HID,), jnp.float32),
            pltpu.VMEM((HID,), jnp.float32),
            pltpu.VMEM((C,), jnp.int32),
            pltpu.VMEM((C,), jnp.int32),
            pltpu.VMEM((C,), jnp.int32),
            pltpu.VMEM((C,), jnp.int32),
            pltpu.VMEM((C,), jnp.int32),
            pltpu.VMEM((C,), jnp.int32),
            pltpu.VMEM((C, HID), jnp.float32),
            pltpu.VMEM((C, HID), jnp.float32),
            pltpu.SemaphoreType.DMA,
            pltpu.SemaphoreType.DMA,
            pltpu.SemaphoreType.DMA,
            pltpu.SemaphoreType.DMA,
            pltpu.SemaphoreType.DMA,
            pltpu.SemaphoreType.DMA,
            pltpu.SemaphoreType.DMA,
            pltpu.SemaphoreType.DMA,
        ],
    )
    def sc_kernel(ids_h, tid_h, kid_h, word_h, tokf_h, taskf_h, w_h, out_h,
                  tok_v, task_v, combo_v, w_v,
                  idx0, idx1, tid0, tid1, kid0, kid1, rows0, rows1,
                  i0, i1, tk0, tk1, g0, g1, o0, o1):
        cid = lax.axis_index("c")
        sid = lax.axis_index("s")
        wid = sid * info.num_cores + cid
        base = wid * tpw

        iota = lax.iota(jnp.int32, L)

        idx_b = (idx0, idx1)
        tid_b = (tid0, tid1)
        kid_b = (kid0, kid1)
        rows_b = (rows0, rows1)
        i_sem = (i0, i1)
        tk_sem = (tk0, tk1)
        g_sem = (g0, g1)
        o_sem = (o0, o1)

        def start_in(tb, b):
            pltpu.async_copy(ids_h.at[pl.ds(tb, C)], idx_b[b], i_sem[b])
            pltpu.async_copy(tid_h.at[pl.ds(tb, C)], tid_b[b], tk_sem[b])
            pltpu.async_copy(kid_h.at[pl.ds(tb, C)], kid_b[b], tk_sem[b])

        def wait_in_ids(b):
            pltpu.make_async_copy(ids_h.at[pl.ds(0, C)], idx_b[b],
                                  i_sem[b]).wait()

        def wait_in_tk(b):
            pltpu.make_async_copy(tid_h.at[pl.ds(0, C)], tid_b[b],
                                  tk_sem[b]).wait()
            pltpu.make_async_copy(kid_h.at[pl.ds(0, C)], kid_b[b],
                                  tk_sem[b]).wait()

        def start_gather(b):
            pltpu.async_copy(word_h.at[idx_b[b]], rows_b[b], g_sem[b])

        def wait_gather(b):
            pltpu.make_async_copy(word_h.at[idx_b[b]], rows_b[b],
                                  g_sem[b]).wait()

        def start_out(tb, b):
            pltpu.async_copy(rows_b[b], out_h.at[pl.ds(tb, C)], o_sem[b])

        def wait_out(b):
            pltpu.make_async_copy(rows_b[b], out_h.at[pl.ds(0, C)],
                                  o_sem[b]).wait()

        def compute_group(b, g):
            rows_v = rows_b[b]
            tid16 = tid_b[b][pl.ds(g * L, L)]
            kid16 = kid_b[b][pl.ds(g * L, L)]
            c16 = tid16 * 16 + kid16
            zero = jnp.zeros((L,), jnp.float32)
            PJ = 8
            sums = []
            for half in range(L // PJ):
                t0 = half * PJ
                cbs = [_lane_splat(c16, t0 + t) * HID + iota
                       for t in range(PJ)]

                @plsc.parallel_loop(0, NJ, unroll=2,
                                    carry=tuple([zero] * PJ))
                def _p1(j, accs):
                    off = j * L
                    combo_s = combo_v.at[pl.ds(off, CSPAN)]
                    out = []
                    for t in range(PJ):
                        w = rows_v[g * L + t0 + t, pl.ds(off, L)]
                        cv = plsc.load_gather(combo_s, [cbs[t]])
                        x = w + cv
                        rows_v[g * L + t0 + t, pl.ds(off, L)] = x
                        out.append(accs[t] + x * x)
                    return tuple(out)

                sums.extend(_p1)

            var = zero
            for t in range(L):
                var = jnp.where(iota == t, jnp.sum(sums[t]), var)
            r16 = _rsqrt16(var * (1.0 / HID) + EPS)

            @plsc.parallel_loop(0, NJ, unroll=2)
            def _scale(j):
                wj = w_v[pl.ds(j * L, L)]
                for t16 in range(L):
                    t = g * L + t16
                    r = _lane_splat(r16, t16)
                    rows_v[t, pl.ds(j * L, L)] = (
                        rows_v[t, pl.ds(j * L, L)] * r * wj)

        start_in(base, 0)
        start_in(base + C, 1)
        pltpu.sync_copy(tokf_h, tok_v)
        pltpu.sync_copy(taskf_h, task_v)
        pltpu.sync_copy(w_h, w_v)
        wait_in_ids(0)
        start_gather(0)

        @plsc.parallel_loop(0, 4 * 16, unroll=2)
        def _build(c):
            bt = (c >> 4) * HID
            bk = (c & 15) * HID
            bc = c * HID
            for j in range(NJ):
                combo_v[pl.ds(bc + j * L, L)] = (
                    tok_v[pl.ds(bt + j * L, L)] + task_v[pl.ds(bk + j * L, L)])

        @pl.loop(0, npair)
        def _pair(p):
            e_tb = base + (2 * p) * C
            o_tb = e_tb + C

            wait_gather(0)
            wait_in_tk(0)
            compute_group(0, 0)

            wait_in_ids(1)

            @pl.when(p > 0)
            def _():
                wait_out(1)

            start_gather(1)
            compute_group(0, 1)
            start_out(e_tb, 0)

            @pl.when(p < npair - 1)
            def _():
                start_in(e_tb + 2 * C, 0)

            wait_gather(1)
            wait_in_tk(1)
            compute_group(1, 0)

            @pl.when(p < npair - 1)
            def _():
                wait_in_ids(0)
                wait_out(0)
                start_gather(0)

            compute_group(1, 1)
            start_out(o_tb, 1)

            @pl.when(p < npair - 1)
            def _():
                start_in(o_tb + 2 * C, 1)

        wait_out(0)
        wait_out(1)

    return sc_kernel


def kernel(input_ids, token_type_ids, task_type_ids, word_table,
           token_type_table, task_type_table, rms_weight):
    b, s = input_ids.shape
    n = b * s
    out = _make_sc_kernel(n)(
        input_ids.reshape(n),
        token_type_ids.reshape(n),
        task_type_ids.reshape(n),
        word_table,
        token_type_table.reshape(-1),
        task_type_table.reshape(-1),
        rms_weight,
    )
    return out.reshape(b, s, HID)
